# baseline (speedup 1.0000x reference)
import jax
import jax.numpy as jnp
from jax import lax
from jax.experimental import pallas as pl
from jax.experimental.pallas import tpu as pltpu
from jax.experimental.pallas import tpu_sc as plsc

_N_ROWS = 600000
_D = 128
_CHUNK = 320
_N_CHUNKS = _N_ROWS // _CHUNK
_NC = 2
_NS = 16
_NW = _NC * _NS
_NBUF = 3
_K_PER_W = 3 * (-(-(-(-_N_CHUNKS // _NW)) // 3))


def _gather_body(nodes_hbm, idx_hbm, out_hbm,
                 idx0, idx1, idx2, rows0, rows1, rows2,
                 g0, g1, g2, s0, s1, s2, i0, i1, i2):
    wid = lax.axis_index("s") * _NC + lax.axis_index("c")
    idx_v = (idx0, idx1, idx2)
    rows_v = (rows0, rows1, rows2)
    gsem = (g0, g1, g2)
    ssem = (s0, s1, s2)
    isem = (i0, i1, i2)

    def start_idx_load(k, b):
        base = (wid + k * _NW) * _CHUNK
        pltpu.async_copy(idx_hbm.at[pl.ds(base, _CHUNK)], idx_v[b], isem[b])

    def wait_scatter(b):
        pltpu.make_async_copy(
            rows_v[b], out_hbm.at[pl.ds(0, _CHUNK)], ssem[b]).wait()

    def start_gather(b):
        pltpu.make_async_copy(
            idx_hbm.at[pl.ds(0, _CHUNK)], idx_v[b], isem[b]).wait()
        pltpu.async_copy(nodes_hbm.at[idx_v[b]], rows_v[b], gsem[b])

    start_idx_load(0, 0)
    start_gather(0)
    start_idx_load(1, 1)

    def step(k, b, b1, b2):
        g_k = wid + k * _NW
        g_n = g_k + _NW

        @pl.when(jnp.logical_and(k >= 2, g_n < _N_CHUNKS))
        def _():
            wait_scatter(b1)

        @pl.when(g_n < _N_CHUNKS)
        def _():
            start_gather(b1)

        @pl.when(g_n + _NW < _N_CHUNKS)
        def _():
            start_idx_load(k + 2, b2)

        @pl.when(g_k < _N_CHUNKS)
        def _():
            pltpu.make_async_copy(
                nodes_hbm.at[idx_v[b]], rows_v[b], gsem[b]).wait()
            pltpu.async_copy(
                rows_v[b], out_hbm.at[pl.ds(g_k * _CHUNK, _CHUNK)], ssem[b])

    def triple(p, carry):
        k0 = 3 * p
        step(k0, 0, 1, 2)
        step(k0 + 1, 1, 2, 0)
        step(k0 + 2, 2, 0, 1)
        return carry

    lax.fori_loop(0, _K_PER_W // 3, triple, 0)
    wait_scatter(0)
    wait_scatter(1)
    wait_scatter(2)


@jax.jit
def kernel(nodes, triangles_indexes):
    t, v = triangles_indexes.shape
    idx = triangles_indexes.astype(jnp.int32).T.reshape(-1)
    mesh = plsc.VectorSubcoreMesh(core_axis_name="c", subcore_axis_name="s")
    gather = pl.kernel(
        _gather_body,
        out_type=jax.ShapeDtypeStruct((_N_ROWS, _D), jnp.float32),
        mesh=mesh,
        scratch_types=[
            pltpu.VMEM((_CHUNK,), jnp.int32),
            pltpu.VMEM((_CHUNK,), jnp.int32),
            pltpu.VMEM((_CHUNK,), jnp.int32),
            pltpu.VMEM((_CHUNK, _D), jnp.float32),
            pltpu.VMEM((_CHUNK, _D), jnp.float32),
            pltpu.VMEM((_CHUNK, _D), jnp.float32),
            pltpu.SemaphoreType.DMA,
            pltpu.SemaphoreType.DMA,
            pltpu.SemaphoreType.DMA,
            pltpu.SemaphoreType.DMA,
            pltpu.SemaphoreType.DMA,
            pltpu.SemaphoreType.DMA,
            pltpu.SemaphoreType.DMA,
            pltpu.SemaphoreType.DMA,
            pltpu.SemaphoreType.DMA,
        ],
    )
    out = gather(nodes, idx)
    return out.reshape(v, t, _D).transpose(1, 0, 2)


# speedup vs baseline: 1.0134x; 1.0134x over previous
import jax
import jax.numpy as jnp
from jax import lax
from jax.experimental import pallas as pl
from jax.experimental.pallas import tpu as pltpu
from jax.experimental.pallas import tpu_sc as plsc

_N_ROWS = 600000
_D = 128
_CHUNK = 240
_N_CHUNKS = _N_ROWS // _CHUNK
_NC = 2
_NS = 16
_NW = _NC * _NS
_NBUF = 4
_K_PER_W = _NBUF * (-(-(-(-_N_CHUNKS // _NW)) // _NBUF))


def _gather_body(nodes_hbm, idx_hbm, out_hbm, *refs):
    idx_v = refs[0:4]
    rows_v = refs[4:8]
    gsem = refs[8:12]
    ssem = refs[12:16]
    isem = refs[16:20]
    wid = lax.axis_index("s") * _NC + lax.axis_index("c")

    def start_idx_load(k, b):
        base = (wid + k * _NW) * _CHUNK
        pltpu.async_copy(idx_hbm.at[pl.ds(base, _CHUNK)], idx_v[b], isem[b])

    def wait_scatter(b):
        pltpu.make_async_copy(
            rows_v[b], out_hbm.at[pl.ds(0, _CHUNK)], ssem[b]).wait()

    def start_gather(b):
        pltpu.make_async_copy(
            idx_hbm.at[pl.ds(0, _CHUNK)], idx_v[b], isem[b]).wait()
        pltpu.async_copy(nodes_hbm.at[idx_v[b]], rows_v[b], gsem[b])

    start_idx_load(0, 0)
    start_gather(0)
    start_idx_load(1, 1)

    def step(k, b, b1, b2):
        g_k = wid + k * _NW
        g_n = g_k + _NW

        @pl.when(jnp.logical_and(k >= _NBUF - 1, g_n < _N_CHUNKS))
        def _():
            wait_scatter(b1)

        @pl.when(g_n < _N_CHUNKS)
        def _():
            start_gather(b1)

        @pl.when(g_n + _NW < _N_CHUNKS)
        def _():
            start_idx_load(k + 2, b2)

        @pl.when(g_k < _N_CHUNKS)
        def _():
            pltpu.make_async_copy(
                nodes_hbm.at[idx_v[b]], rows_v[b], gsem[b]).wait()
            pltpu.async_copy(
                rows_v[b], out_hbm.at[pl.ds(g_k * _CHUNK, _CHUNK)], ssem[b])

    def quad(p, carry):
        k0 = _NBUF * p
        for j in range(_NBUF):
            step(k0 + j, j, (j + 1) % _NBUF, (j + 2) % _NBUF)
        return carry

    lax.fori_loop(0, _K_PER_W // _NBUF, quad, 0)
    for b in range(_NBUF):
        wait_scatter(b)


@jax.jit
def kernel(nodes, triangles_indexes):
    t, v = triangles_indexes.shape
    idx = triangles_indexes.astype(jnp.int32).T.reshape(-1)
    mesh = plsc.VectorSubcoreMesh(core_axis_name="c", subcore_axis_name="s")
    gather = pl.kernel(
        _gather_body,
        out_type=jax.ShapeDtypeStruct((_N_ROWS, _D), jnp.float32),
        mesh=mesh,
        scratch_types=(
            [pltpu.VMEM((_CHUNK,), jnp.int32)] * _NBUF
            + [pltpu.VMEM((_CHUNK, _D), jnp.float32)] * _NBUF
            + [pltpu.SemaphoreType.DMA] * (3 * _NBUF)
        ),
    )
    out = gather(nodes, idx)
    return out.reshape(v, t, _D).transpose(1, 0, 2)


# speedup vs baseline: 1.0158x; 1.0023x over previous
import jax
import jax.numpy as jnp
from jax import lax
from jax.experimental import pallas as pl
from jax.experimental.pallas import tpu as pltpu
from jax.experimental.pallas import tpu_sc as plsc

_N_ROWS = 600000
_D = 128
_CHUNK = 160
_N_CHUNKS = _N_ROWS // _CHUNK
_NC = 2
_NS = 16
_NW = _NC * _NS
_NBUF = 6
_K_PER_W = _NBUF * (-(-(-(-_N_CHUNKS // _NW)) // _NBUF))


def _gather_body(nodes_hbm, idx_hbm, out_hbm, *refs):
    idx_v = refs[0:_NBUF]
    rows_v = refs[_NBUF:2 * _NBUF]
    gsem = refs[2 * _NBUF:3 * _NBUF]
    ssem = refs[3 * _NBUF:4 * _NBUF]
    isem = refs[4 * _NBUF:5 * _NBUF]
    wid = lax.axis_index("s") * _NC + lax.axis_index("c")

    def start_idx_load(k, b):
        base = (wid + k * _NW) * _CHUNK
        pltpu.async_copy(idx_hbm.at[pl.ds(base, _CHUNK)], idx_v[b], isem[b])

    def wait_scatter(b):
        pltpu.make_async_copy(
            rows_v[b], out_hbm.at[pl.ds(0, _CHUNK)], ssem[b]).wait()

    def start_gather(b):
        pltpu.make_async_copy(
            idx_hbm.at[pl.ds(0, _CHUNK)], idx_v[b], isem[b]).wait()
        pltpu.async_copy(nodes_hbm.at[idx_v[b]], rows_v[b], gsem[b])

    start_idx_load(0, 0)
    start_gather(0)
    start_idx_load(1, 1)

    def step(k, b, b1, b2):
        g_k = wid + k * _NW
        g_n = g_k + _NW

        @pl.when(jnp.logical_and(k >= _NBUF - 1, g_n < _N_CHUNKS))
        def _():
            wait_scatter(b1)

        @pl.when(g_n < _N_CHUNKS)
        def _():
            start_gather(b1)

        @pl.when(g_n + _NW < _N_CHUNKS)
        def _():
            start_idx_load(k + 2, b2)

        @pl.when(g_k < _N_CHUNKS)
        def _():
            pltpu.make_async_copy(
                nodes_hbm.at[idx_v[b]], rows_v[b], gsem[b]).wait()
            pltpu.async_copy(
                rows_v[b], out_hbm.at[pl.ds(g_k * _CHUNK, _CHUNK)], ssem[b])

    def quad(p, carry):
        k0 = _NBUF * p
        for j in range(_NBUF):
            step(k0 + j, j, (j + 1) % _NBUF, (j + 2) % _NBUF)
        return carry

    lax.fori_loop(0, _K_PER_W // _NBUF, quad, 0)
    for b in range(_NBUF):
        wait_scatter(b)


@jax.jit
def kernel(nodes, triangles_indexes):
    t, v = triangles_indexes.shape
    idx = triangles_indexes.astype(jnp.int32).T.reshape(-1)
    mesh = plsc.VectorSubcoreMesh(core_axis_name="c", subcore_axis_name="s")
    gather = pl.kernel(
        _gather_body,
        out_type=jax.ShapeDtypeStruct((_N_ROWS, _D), jnp.float32),
        mesh=mesh,
        scratch_types=(
            [pltpu.VMEM((_CHUNK,), jnp.int32)] * _NBUF
            + [pltpu.VMEM((_CHUNK, _D), jnp.float32)] * _NBUF
            + [pltpu.SemaphoreType.DMA] * (3 * _NBUF)
        ),
    )
    out = gather(nodes, idx)
    return out.reshape(v, t, _D).transpose(1, 0, 2)
